# Initial kernel scaffold; baseline (speedup 1.0000x reference)
import jax
import jax.numpy as jnp
from jax.experimental import pallas as pl


def kernel(coords, codebook_0, codebook_1, codebook_2, codebook_3, codebook_4, codebook_5, codebook_6, codebook_7):
    raise NotImplementedError("write your pallas kernel here")


# speedup vs baseline: 46.4541x; 46.4541x over previous
import functools

import numpy as np
import jax
import jax.numpy as jnp
from jax import lax
from jax.experimental import pallas as pl
from jax.experimental.pallas import tpu as pltpu
from jax.experimental.pallas import tpu_sc as plsc

BAND_WIDTH = 16
MIN_GRID_RES = 16
MAX_GRID_RES = 512
NUM_LOD = 8
CODEBOOK_SIZE = 2 ** BAND_WIDTH
N_POINTS = 1048576
_b = float(np.exp((np.log(MAX_GRID_RES) - np.log(MIN_GRID_RES)) / (NUM_LOD - 1)))
LODS = [int(1 + np.floor(MIN_GRID_RES * _b ** l)) for l in range(NUM_LOD)]
SMALL_LODS = [l for l in LODS if l * l <= CODEBOOK_SIZE]
BIG_LODS = [l for l in LODS if l * l > CODEBOOK_SIZE]
N_SMALL = len(SMALL_LODS)
N_BIG = len(BIG_LODS)
PRIME1_I32 = -1640531535
HASH_MASK = CODEBOOK_SIZE - 1

SR = 4
BIG_ROWS = CODEBOOK_SIZE // SR
NC, NS, LANES = 2, 16, 16
NW = NC * NS
PPW = N_POINTS // NW
C = 128
NCH = PPW // C
G = C // LANES

_mesh = plsc.VectorSubcoreMesh(core_axis_name="c", subcore_axis_name="s")

_scratch = (
    [pltpu.VMEM((2 * l * l,), jnp.float32) for l in SMALL_LODS]
    + [pltpu.VMEM((2 * C,), jnp.float32)]
    + [pltpu.VMEM((16 * C,), jnp.float32)]
    + [pltpu.VMEM((C,), jnp.int32) for _ in range(4 * N_BIG)]
    + [pltpu.VMEM((C, 2 * SR), jnp.float32) for _ in range(4 * N_BIG)]
    + [pltpu.SemaphoreType.DMA]
)


def _corners(xi, yi, lod):
    s = jnp.float32(lod - 1)
    xs = xi * s
    ys = yi * s
    x0 = xs.astype(jnp.int32)
    y0 = ys.astype(jnp.int32)
    wx = xs - x0.astype(jnp.float32)
    wy = ys - y0.astype(jnp.float32)
    x1 = jnp.minimum(x0 + 1, lod - 1)
    y1 = jnp.minimum(y0 + 1, lod - 1)
    return x0, x1, y0, y1, wx, wy


def _blend(c00, c10, c01, c11, wx, wy):
    f0 = c00 + wx * (c10 - c00)
    f1 = c01 + wx * (c11 - c01)
    return f0 + wy * (f1 - f0)


def _sc_body(*refs):
    coords_f = refs[0]
    cb_small_hbm = refs[1:1 + N_SMALL]
    cb_big_hbm = refs[1 + N_SMALL:1 + NUM_LOD]
    out_f = refs[1 + NUM_LOD]
    scr = refs[2 + NUM_LOD:]
    cbv = scr[0:N_SMALL]
    xv = scr[N_SMALL]
    ov = scr[N_SMALL + 1]
    idxr = scr[N_SMALL + 2:N_SMALL + 2 + 4 * N_BIG]
    rowr = scr[N_SMALL + 2 + 4 * N_BIG:N_SMALL + 2 + 8 * N_BIG]
    sem = scr[N_SMALL + 2 + 8 * N_BIG]

    wid = lax.axis_index("s") * NC + lax.axis_index("c")
    for i in range(N_SMALL):
        pltpu.sync_copy(cb_small_hbm[i], cbv[i])

    base0 = wid * PPW
    lane = lax.iota(jnp.int32, LANES)
    z16 = lane * 0
    o16 = z16 + 1

    def load_xy(g):
        b = g * (2 * LANES) + lane * 2
        xi = plsc.load_gather(xv, [b])
        yi = plsc.load_gather(xv, [b + 1])
        return xi, yi

    def chunk_body(ci, carry):
        base = base0 + ci * C
        pltpu.sync_copy(coords_f.at[pl.ds(base * 2, 2 * C)], xv)

        def idx_body(g, cr):
            xi, yi = load_xy(g)
            sl = pl.ds(g * LANES, LANES)
            for k, lod in enumerate(BIG_LODS):
                x0, x1, y0, y1, _, _ = _corners(xi, yi, lod)
                t0 = y0 * PRIME1_I32
                t1 = y1 * PRIME1_I32
                idxr[4 * k + 0][sl] = ((x0 ^ t0) & HASH_MASK) >> 2
                idxr[4 * k + 1][sl] = ((x1 ^ t0) & HASH_MASK) >> 2
                idxr[4 * k + 2][sl] = ((x0 ^ t1) & HASH_MASK) >> 2
                idxr[4 * k + 3][sl] = ((x1 ^ t1) & HASH_MASK) >> 2
            return cr

        lax.fori_loop(0, G, idx_body, 0)

        copies = []
        for k in range(N_BIG):
            for c4 in range(4):
                copies.append(
                    pltpu.async_copy(cb_big_hbm[k].at[idxr[4 * k + c4]],
                                     rowr[4 * k + c4], sem))

        def small_body(g, cr):
            xi, yi = load_xy(g)
            oidx = g * (16 * LANES) + lane * 16
            for li, lod in enumerate(SMALL_LODS):
                x0, x1, y0, y1, wx, wy = _corners(xi, yi, lod)
                b0 = y0 * lod
                b1 = y1 * lod
                i00 = (x0 + b0) * 2
                i10 = (x1 + b0) * 2
                i01 = (x0 + b1) * 2
                i11 = (x1 + b1) * 2
                cb = cbv[li]
                c00a = plsc.load_gather(cb, [i00])
                c00b = plsc.load_gather(cb, [i00 + 1])
                c10a = plsc.load_gather(cb, [i10])
                c10b = plsc.load_gather(cb, [i10 + 1])
                c01a = plsc.load_gather(cb, [i01])
                c01b = plsc.load_gather(cb, [i01 + 1])
                c11a = plsc.load_gather(cb, [i11])
                c11b = plsc.load_gather(cb, [i11 + 1])
                fa = _blend(c00a, c10a, c01a, c11a, wx, wy)
                fb = _blend(c00b, c10b, c01b, c11b, wx, wy)
                plsc.store_scatter(ov, [oidx + (2 * li)], fa)
                plsc.store_scatter(ov, [oidx + (2 * li + 1)], fb)
            return cr

        lax.fori_loop(0, G, small_body, 0)

        for cp in copies:
            cp.wait()

        def big_body(g, cr):
            xi, yi = load_xy(g)
            oidx = g * (16 * LANES) + lane * 16
            ridx = g * LANES + lane
            for k, lod in enumerate(BIG_LODS):
                x0, x1, y0, y1, wx, wy = _corners(xi, yi, lod)
                t0 = y0 * PRIME1_I32
                t1 = y1 * PRIME1_I32
                s00 = (((x0 ^ t0) & HASH_MASK) & (SR - 1)) * 2
                s10 = (((x1 ^ t0) & HASH_MASK) & (SR - 1)) * 2
                s01 = (((x0 ^ t1) & HASH_MASK) & (SR - 1)) * 2
                s11 = (((x1 ^ t1) & HASH_MASK) & (SR - 1)) * 2
                r00, r10, r01, r11 = (rowr[4 * k + j] for j in range(4))
                c00a = plsc.load_gather(r00, [ridx, s00])
                c00b = plsc.load_gather(r00, [ridx, s00 + 1])
                c10a = plsc.load_gather(r10, [ridx, s10])
                c10b = plsc.load_gather(r10, [ridx, s10 + 1])
                c01a = plsc.load_gather(r01, [ridx, s01])
                c01b = plsc.load_gather(r01, [ridx, s01 + 1])
                c11a = plsc.load_gather(r11, [ridx, s11])
                c11b = plsc.load_gather(r11, [ridx, s11 + 1])
                fa = _blend(c00a, c10a, c01a, c11a, wx, wy)
                fb = _blend(c00b, c10b, c01b, c11b, wx, wy)
                col = 2 * (N_SMALL + k)
                plsc.store_scatter(ov, [oidx + col], fa)
                plsc.store_scatter(ov, [oidx + (col + 1)], fb)
            return cr

        lax.fori_loop(0, G, big_body, 0)

        pltpu.sync_copy(ov, out_f.at[pl.ds(base * 16, 16 * C)])
        return carry

    lax.fori_loop(0, NCH, chunk_body, 0)


_hash_grid_sc = functools.partial(
    pl.kernel,
    mesh=_mesh,
    out_type=jax.ShapeDtypeStruct((N_POINTS * 16,), jnp.float32),
    scratch_types=_scratch,
    compiler_params=pltpu.CompilerParams(needs_layout_passes=False,
                                         use_tc_tiling_on_sc=False),
)()(_sc_body)


def kernel(coords, codebook_0, codebook_1, codebook_2, codebook_3,
           codebook_4, codebook_5, codebook_6, codebook_7):
    cbs = (codebook_0, codebook_1, codebook_2, codebook_3,
           codebook_4, codebook_5, codebook_6, codebook_7)
    small = [cb.reshape(-1) for cb, l in zip(cbs, LODS) if l * l <= CODEBOOK_SIZE]
    big = [cb.reshape(BIG_ROWS, 2 * SR)
           for cb, l in zip(cbs, LODS) if l * l > CODEBOOK_SIZE]
    out = _hash_grid_sc(coords.reshape(-1), *small, *big)
    return out.reshape(N_POINTS, 16)


# speedup vs baseline: 49.6807x; 1.0695x over previous
import functools

import numpy as np
import jax
import jax.numpy as jnp
from jax import lax
from jax.experimental import pallas as pl
from jax.experimental.pallas import tpu as pltpu
from jax.experimental.pallas import tpu_sc as plsc

BAND_WIDTH = 16
MIN_GRID_RES = 16
MAX_GRID_RES = 512
NUM_LOD = 8
CODEBOOK_SIZE = 2 ** BAND_WIDTH
N_POINTS = 1048576
_b = float(np.exp((np.log(MAX_GRID_RES) - np.log(MIN_GRID_RES)) / (NUM_LOD - 1)))
LODS = [int(1 + np.floor(MIN_GRID_RES * _b ** l)) for l in range(NUM_LOD)]
SMALL_LODS = [l for l in LODS if l * l <= CODEBOOK_SIZE]
BIG_LODS = [l for l in LODS if l * l > CODEBOOK_SIZE]
N_SMALL = len(SMALL_LODS)
N_BIG = len(BIG_LODS)
PRIME1_I32 = -1640531535
HASH_MASK = CODEBOOK_SIZE - 1

SR = 4
BIG_ROWS = CODEBOOK_SIZE // SR
NC, NS, LANES = 2, 16, 16
NW = NC * NS
PPW = N_POINTS // NW
C = 128
NCH = PPW // C
G = C // LANES

_mesh = plsc.VectorSubcoreMesh(core_axis_name="c", subcore_axis_name="s")

_scratch = (
    [pltpu.VMEM((2 * l * l,), jnp.float32) for l in SMALL_LODS]
    + [pltpu.VMEM((C,), jnp.float32)]
    + [pltpu.VMEM((C,), jnp.float32)]
    + [pltpu.VMEM((16 * C,), jnp.float32)]
    + [pltpu.VMEM((C,), jnp.int32) for _ in range(4 * N_BIG)]
    + [pltpu.VMEM((C, 2 * SR), jnp.float32) for _ in range(4 * N_BIG)]
    + [pltpu.SemaphoreType.DMA, pltpu.SemaphoreType.DMA]
)


def _corners(xi, yi, lod):
    s = jnp.float32(lod - 1)
    xs = xi * s
    ys = yi * s
    x0 = xs.astype(jnp.int32)
    y0 = ys.astype(jnp.int32)
    wx = xs - x0.astype(jnp.float32)
    wy = ys - y0.astype(jnp.float32)
    x1 = jnp.minimum(x0 + 1, lod - 1)
    y1 = jnp.minimum(y0 + 1, lod - 1)
    return x0, x1, y0, y1, wx, wy


def _blend(c00, c10, c01, c11, wx, wy):
    f0 = c00 + wx * (c10 - c00)
    f1 = c01 + wx * (c11 - c01)
    return f0 + wy * (f1 - f0)


def _sc_body(*refs):
    xs_hbm = refs[0]
    ys_hbm = refs[1]
    cb_small_hbm = refs[2:2 + N_SMALL]
    cb_big_hbm = refs[2 + N_SMALL:2 + NUM_LOD]
    out_f = refs[2 + NUM_LOD]
    scr = refs[3 + NUM_LOD:]
    cbv = scr[0:N_SMALL]
    xv = scr[N_SMALL]
    yv = scr[N_SMALL + 1]
    ov = scr[N_SMALL + 2]
    idxr = scr[N_SMALL + 3:N_SMALL + 3 + 4 * N_BIG]
    rowr = scr[N_SMALL + 3 + 4 * N_BIG:N_SMALL + 3 + 8 * N_BIG]
    sem = scr[N_SMALL + 3 + 8 * N_BIG]
    osem = scr[N_SMALL + 4 + 8 * N_BIG]

    sid = lax.axis_index("s")
    wid = sid * NC + lax.axis_index("c")
    for i in range(N_SMALL):
        pltpu.sync_copy(cb_small_hbm[i], cbv[i])

    base0 = wid * PPW
    lane = lax.iota(jnp.int32, LANES)
    z16 = lane * 0
    o16 = z16 + 1

    def load_xy(g):
        xi = xv[pl.ds(g * LANES, LANES)]
        yi = yv[pl.ds(g * LANES, LANES)]
        return xi, yi

    def chunk_body(ci, carry):
        base = base0 + ci * C
        pltpu.sync_copy(xs_hbm.at[pl.ds(base, C)], xv)
        pltpu.sync_copy(ys_hbm.at[pl.ds(base, C)], yv)

        def idx_body(g, cr):
            xi, yi = load_xy(g)
            sl = pl.ds(g * LANES, LANES)
            for k, lod in enumerate(BIG_LODS):
                x0, x1, y0, y1, _, _ = _corners(xi, yi, lod)
                t0 = y0 * PRIME1_I32
                t1 = y1 * PRIME1_I32
                idxr[4 * k + 0][sl] = ((x0 ^ t0) & HASH_MASK) >> 2
                idxr[4 * k + 1][sl] = ((x1 ^ t0) & HASH_MASK) >> 2
                idxr[4 * k + 2][sl] = ((x0 ^ t1) & HASH_MASK) >> 2
                idxr[4 * k + 3][sl] = ((x1 ^ t1) & HASH_MASK) >> 2
            return cr

        lax.fori_loop(0, G, idx_body, 0)

        copies = []
        for k in range(N_BIG):
            for c4 in range(4):
                copies.append(
                    pltpu.async_copy(cb_big_hbm[k].at[idxr[4 * k + c4]],
                                     rowr[4 * k + c4], sem))

        def small_body(g, cr):
            xi, yi = load_xy(g)
            o = g * LANES
            for li, lod in enumerate(SMALL_LODS):
                x0, x1, y0, y1, wx, wy = _corners(xi, yi, lod)
                b0 = y0 * lod
                b1 = y1 * lod
                i00 = (x0 + b0) * 2
                i10 = (x1 + b0) * 2
                i01 = (x0 + b1) * 2
                i11 = (x1 + b1) * 2
                cb = cbv[li]
                c00a = plsc.load_gather(cb, [i00])
                c00b = plsc.load_gather(cb, [i00 + 1])
                c10a = plsc.load_gather(cb, [i10])
                c10b = plsc.load_gather(cb, [i10 + 1])
                c01a = plsc.load_gather(cb, [i01])
                c01b = plsc.load_gather(cb, [i01 + 1])
                c11a = plsc.load_gather(cb, [i11])
                c11b = plsc.load_gather(cb, [i11 + 1])
                fa = _blend(c00a, c10a, c01a, c11a, wx, wy)
                fb = _blend(c00b, c10b, c01b, c11b, wx, wy)
                ov[pl.ds((2 * li) * C + o, LANES)] = fa
                ov[pl.ds((2 * li + 1) * C + o, LANES)] = fb
            return cr

        lax.fori_loop(0, G, small_body, 0)

        for cp in copies:
            cp.wait()

        def big_body(g, cr):
            xi, yi = load_xy(g)
            o = g * LANES
            ridx = g * LANES + lane
            for k, lod in enumerate(BIG_LODS):
                x0, x1, y0, y1, wx, wy = _corners(xi, yi, lod)
                t0 = y0 * PRIME1_I32
                t1 = y1 * PRIME1_I32
                s00 = (((x0 ^ t0) & HASH_MASK) & (SR - 1)) * 2
                s10 = (((x1 ^ t0) & HASH_MASK) & (SR - 1)) * 2
                s01 = (((x0 ^ t1) & HASH_MASK) & (SR - 1)) * 2
                s11 = (((x1 ^ t1) & HASH_MASK) & (SR - 1)) * 2
                r00, r10, r01, r11 = (rowr[4 * k + j] for j in range(4))
                c00a = plsc.load_gather(r00, [ridx, s00])
                c00b = plsc.load_gather(r00, [ridx, s00 + 1])
                c10a = plsc.load_gather(r10, [ridx, s10])
                c10b = plsc.load_gather(r10, [ridx, s10 + 1])
                c01a = plsc.load_gather(r01, [ridx, s01])
                c01b = plsc.load_gather(r01, [ridx, s01 + 1])
                c11a = plsc.load_gather(r11, [ridx, s11])
                c11b = plsc.load_gather(r11, [ridx, s11 + 1])
                fa = _blend(c00a, c10a, c01a, c11a, wx, wy)
                fb = _blend(c00b, c10b, c01b, c11b, wx, wy)
                li = N_SMALL + k
                ov[pl.ds((2 * li) * C + o, LANES)] = fa
                ov[pl.ds((2 * li + 1) * C + o, LANES)] = fb
            return cr

        lax.fori_loop(0, G, big_body, 0)

        outs = [pltpu.async_copy(ov.at[pl.ds(j * C, C)],
                                 out_f.at[pl.ds(j * N_POINTS + base, C)], osem)
                for j in range(16)]
        for cp in outs:
            cp.wait()
        return carry

    lax.fori_loop(0, NCH, chunk_body, 0)


_hash_grid_sc = functools.partial(
    pl.kernel,
    mesh=_mesh,
    out_type=jax.ShapeDtypeStruct((N_POINTS * 16,), jnp.float32),
    scratch_types=_scratch,
    compiler_params=pltpu.CompilerParams(needs_layout_passes=False,
                                         use_tc_tiling_on_sc=False),
)()(_sc_body)


def kernel(coords, codebook_0, codebook_1, codebook_2, codebook_3,
           codebook_4, codebook_5, codebook_6, codebook_7):
    cbs = (codebook_0, codebook_1, codebook_2, codebook_3,
           codebook_4, codebook_5, codebook_6, codebook_7)
    small = [cb.reshape(-1) for cb, l in zip(cbs, LODS) if l * l <= CODEBOOK_SIZE]
    big = [cb.reshape(BIG_ROWS, 2 * SR)
           for cb, l in zip(cbs, LODS) if l * l > CODEBOOK_SIZE]
    out = _hash_grid_sc(coords[:, 0], coords[:, 1], *small, *big)
    return out.reshape(16, N_POINTS).T


# speedup vs baseline: 117.6940x; 2.3690x over previous
import functools

import numpy as np
import jax
import jax.numpy as jnp
from jax import lax
from jax.experimental import pallas as pl
from jax.experimental.pallas import tpu as pltpu
from jax.experimental.pallas import tpu_sc as plsc

BAND_WIDTH = 16
MIN_GRID_RES = 16
MAX_GRID_RES = 512
NUM_LOD = 8
CODEBOOK_SIZE = 2 ** BAND_WIDTH
N_POINTS = 1048576
_b = float(np.exp((np.log(MAX_GRID_RES) - np.log(MIN_GRID_RES)) / (NUM_LOD - 1)))
LODS = [int(1 + np.floor(MIN_GRID_RES * _b ** l)) for l in range(NUM_LOD)]
SMALL_LODS = [l for l in LODS if l * l <= CODEBOOK_SIZE]
BIG_LODS = [l for l in LODS if l * l > CODEBOOK_SIZE]
N_SMALL = len(SMALL_LODS)
N_BIG = len(BIG_LODS)
PRIME1_I32 = -1640531535
HASH_MASK = CODEBOOK_SIZE - 1

SR = 4
BIG_ROWS = CODEBOOK_SIZE // SR
NC, NS, LANES = 2, 16, 16
NW = NC * NS
PPW = N_POINTS // NW
C = 128
NCH = PPW // C
G = C // LANES

_mesh = plsc.VectorSubcoreMesh(core_axis_name="c", subcore_axis_name="s")

_scratch = (
    [pltpu.VMEM((2 * l * l,), jnp.float32) for l in SMALL_LODS]
    + [pltpu.VMEM((C,), jnp.float32)]
    + [pltpu.VMEM((C,), jnp.float32)]
    + [pltpu.VMEM((16 * C,), jnp.float32)]
    + [pltpu.VMEM((C,), jnp.int32) for _ in range(4 * N_BIG)]
    + [pltpu.VMEM((C, 2 * SR), jnp.float32) for _ in range(4 * N_BIG)]
    + [pltpu.SemaphoreType.DMA, pltpu.SemaphoreType.DMA]
)


def _corners(xi, yi, lod):
    s = jnp.float32(lod - 1)
    xs = xi * s
    ys = yi * s
    x0 = xs.astype(jnp.int32)
    y0 = ys.astype(jnp.int32)
    wx = xs - x0.astype(jnp.float32)
    wy = ys - y0.astype(jnp.float32)
    x1 = jnp.minimum(x0 + 1, lod - 1)
    y1 = jnp.minimum(y0 + 1, lod - 1)
    return x0, x1, y0, y1, wx, wy


def _blend(c00, c10, c01, c11, wx, wy):
    f0 = c00 + wx * (c10 - c00)
    f1 = c01 + wx * (c11 - c01)
    return f0 + wy * (f1 - f0)


def _sc_body(*refs):
    xs_hbm = refs[0]
    ys_hbm = refs[1]
    cb_small_hbm = refs[2:2 + N_SMALL]
    cb_big_hbm = refs[2 + N_SMALL:2 + NUM_LOD]
    out_f = refs[2 + NUM_LOD]
    scr = refs[3 + NUM_LOD:]
    cbv = scr[0:N_SMALL]
    xv = scr[N_SMALL]
    yv = scr[N_SMALL + 1]
    ov = scr[N_SMALL + 2]
    idxr = scr[N_SMALL + 3:N_SMALL + 3 + 4 * N_BIG]
    rowr = scr[N_SMALL + 3 + 4 * N_BIG:N_SMALL + 3 + 8 * N_BIG]
    sem = scr[N_SMALL + 3 + 8 * N_BIG]
    osem = scr[N_SMALL + 4 + 8 * N_BIG]

    sid = lax.axis_index("s")
    wid = sid * NC + lax.axis_index("c")
    for i in range(N_SMALL):
        pltpu.sync_copy(cb_small_hbm[i], cbv[i])

    base0 = wid * PPW
    lane = lax.iota(jnp.int32, LANES)
    z16 = lane * 0
    o16 = z16 + 1

    def load_xy(g):
        xi = xv[pl.ds(g * LANES, LANES)]
        yi = yv[pl.ds(g * LANES, LANES)]
        return xi, yi

    def chunk_body(ci, carry):
        base = base0 + ci * C
        pltpu.sync_copy(xs_hbm.at[pl.ds(base, C)], xv)
        pltpu.sync_copy(ys_hbm.at[pl.ds(base, C)], yv)

        def idx_body(g, cr):
            xi, yi = load_xy(g)
            sl = pl.ds(g * LANES, LANES)
            for k, lod in enumerate(BIG_LODS):
                x0, x1, y0, y1, _, _ = _corners(xi, yi, lod)
                t0 = y0 * PRIME1_I32
                t1 = y1 * PRIME1_I32
                idxr[4 * k + 0][sl] = ((x0 ^ t0) & HASH_MASK) >> 2
                idxr[4 * k + 1][sl] = ((x1 ^ t0) & HASH_MASK) >> 2
                idxr[4 * k + 2][sl] = ((x0 ^ t1) & HASH_MASK) >> 2
                idxr[4 * k + 3][sl] = ((x1 ^ t1) & HASH_MASK) >> 2
            return cr

        lax.fori_loop(0, G, idx_body, 0)

        copies = []
        for k in range(N_BIG):
            for c4 in range(4):
                copies.append(
                    pltpu.async_copy(cb_big_hbm[k].at[idxr[4 * k + c4]],
                                     rowr[4 * k + c4], sem))

        def small_body(g, cr):
            xi, yi = load_xy(g)
            o = g * LANES
            for li, lod in enumerate(SMALL_LODS):
                x0, x1, y0, y1, wx, wy = _corners(xi, yi, lod)
                b0 = y0 * lod
                b1 = y1 * lod
                i00 = (x0 + b0) * 2
                i10 = (x1 + b0) * 2
                i01 = (x0 + b1) * 2
                i11 = (x1 + b1) * 2
                cb = cbv[li]
                c00a = plsc.load_gather(cb, [i00])
                c00b = plsc.load_gather(cb, [i00 + 1])
                c10a = plsc.load_gather(cb, [i10])
                c10b = plsc.load_gather(cb, [i10 + 1])
                c01a = plsc.load_gather(cb, [i01])
                c01b = plsc.load_gather(cb, [i01 + 1])
                c11a = plsc.load_gather(cb, [i11])
                c11b = plsc.load_gather(cb, [i11 + 1])
                fa = _blend(c00a, c10a, c01a, c11a, wx, wy)
                fb = _blend(c00b, c10b, c01b, c11b, wx, wy)
                ov[pl.ds((2 * li) * C + o, LANES)] = fa
                ov[pl.ds((2 * li + 1) * C + o, LANES)] = fb
            return cr

        lax.fori_loop(0, G, small_body, 0)

        for cp in copies:
            cp.wait()

        def big_body(g, cr):
            xi, yi = load_xy(g)
            o = g * LANES
            ridx = g * LANES + lane
            for k, lod in enumerate(BIG_LODS):
                x0, x1, y0, y1, wx, wy = _corners(xi, yi, lod)
                t0 = y0 * PRIME1_I32
                t1 = y1 * PRIME1_I32
                s00 = (((x0 ^ t0) & HASH_MASK) & (SR - 1)) * 2
                s10 = (((x1 ^ t0) & HASH_MASK) & (SR - 1)) * 2
                s01 = (((x0 ^ t1) & HASH_MASK) & (SR - 1)) * 2
                s11 = (((x1 ^ t1) & HASH_MASK) & (SR - 1)) * 2
                r00, r10, r01, r11 = (rowr[4 * k + j] for j in range(4))
                c00a = plsc.load_gather(r00, [ridx, s00])
                c00b = plsc.load_gather(r00, [ridx, s00 + 1])
                c10a = plsc.load_gather(r10, [ridx, s10])
                c10b = plsc.load_gather(r10, [ridx, s10 + 1])
                c01a = plsc.load_gather(r01, [ridx, s01])
                c01b = plsc.load_gather(r01, [ridx, s01 + 1])
                c11a = plsc.load_gather(r11, [ridx, s11])
                c11b = plsc.load_gather(r11, [ridx, s11 + 1])
                fa = _blend(c00a, c10a, c01a, c11a, wx, wy)
                fb = _blend(c00b, c10b, c01b, c11b, wx, wy)
                li = N_SMALL + k
                ov[pl.ds((2 * li) * C + o, LANES)] = fa
                ov[pl.ds((2 * li + 1) * C + o, LANES)] = fb
            return cr

        lax.fori_loop(0, G, big_body, 0)

        half = (N_POINTS // C) * (8 * C)
        cpa = pltpu.async_copy(ov.at[pl.ds(0, 8 * C)],
                               out_f.at[pl.ds(base * 8, 8 * C)], osem)
        cpb = pltpu.async_copy(ov.at[pl.ds(8 * C, 8 * C)],
                               out_f.at[pl.ds(half + base * 8, 8 * C)], osem)
        cpa.wait()
        cpb.wait()
        return carry

    lax.fori_loop(0, NCH, chunk_body, 0)


_hash_grid_sc = functools.partial(
    pl.kernel,
    mesh=_mesh,
    out_type=jax.ShapeDtypeStruct((N_POINTS * 16,), jnp.float32),
    scratch_types=_scratch,
    compiler_params=pltpu.CompilerParams(needs_layout_passes=False,
                                         use_tc_tiling_on_sc=False),
)()(_sc_body)


def kernel(coords, codebook_0, codebook_1, codebook_2, codebook_3,
           codebook_4, codebook_5, codebook_6, codebook_7):
    cbs = (codebook_0, codebook_1, codebook_2, codebook_3,
           codebook_4, codebook_5, codebook_6, codebook_7)
    small = [cb.reshape(-1) for cb, l in zip(cbs, LODS) if l * l <= CODEBOOK_SIZE]
    big = [cb.reshape(BIG_ROWS, 2 * SR)
           for cb, l in zip(cbs, LODS) if l * l > CODEBOOK_SIZE]
    out = _hash_grid_sc(coords[:, 0], coords[:, 1], *small, *big)
    nb = N_POINTS // C
    return (out.reshape(2, nb, 8, C).transpose(1, 3, 0, 2)
            .reshape(N_POINTS, 16))


# speedup vs baseline: 135.6583x; 1.1526x over previous
import functools

import numpy as np
import jax
import jax.numpy as jnp
from jax import lax
from jax.experimental import pallas as pl
from jax.experimental.pallas import tpu as pltpu
from jax.experimental.pallas import tpu_sc as plsc

BAND_WIDTH = 16
MIN_GRID_RES = 16
MAX_GRID_RES = 512
NUM_LOD = 8
CODEBOOK_SIZE = 2 ** BAND_WIDTH
N_POINTS = 1048576
_b = float(np.exp((np.log(MAX_GRID_RES) - np.log(MIN_GRID_RES)) / (NUM_LOD - 1)))
LODS = [int(1 + np.floor(MIN_GRID_RES * _b ** l)) for l in range(NUM_LOD)]
SMALL_LODS = [l for l in LODS if l * l <= CODEBOOK_SIZE]
BIG_LODS = [l for l in LODS if l * l > CODEBOOK_SIZE]
N_SMALL = len(SMALL_LODS)
N_BIG = len(BIG_LODS)
PRIME1_I32 = -1640531535
HASH_MASK = CODEBOOK_SIZE - 1

SR = 4
BIG_ROWS = CODEBOOK_SIZE // SR
NC, NS, LANES = 2, 16, 16
NW = NC * NS
PPW = N_POINTS // NW
C = 128
NCH = PPW // C
G = C // LANES

_mesh = plsc.VectorSubcoreMesh(core_axis_name="c", subcore_axis_name="s")

_scratch = (
    [pltpu.VMEM((2 * l * l,), jnp.float32) for l in SMALL_LODS]
    + [pltpu.VMEM((2 * C,), jnp.float32)]
    + [pltpu.VMEM((2 * C,), jnp.float32)]
    + [pltpu.VMEM((16 * C,), jnp.float32)]
    + [pltpu.VMEM((16 * C,), jnp.float32)]
    + [pltpu.VMEM((C,), jnp.int32) for _ in range(4 * N_BIG)]
    + [pltpu.VMEM((C, 2 * SR), jnp.float32) for _ in range(4 * N_BIG)]
    + [pltpu.SemaphoreType.DMA, pltpu.SemaphoreType.DMA]
)


def _corners(xi, yi, lod):
    s = jnp.float32(lod - 1)
    xs = xi * s
    ys = yi * s
    x0 = xs.astype(jnp.int32)
    y0 = ys.astype(jnp.int32)
    wx = xs - x0.astype(jnp.float32)
    wy = ys - y0.astype(jnp.float32)
    x1 = jnp.minimum(x0 + 1, lod - 1)
    y1 = jnp.minimum(y0 + 1, lod - 1)
    return x0, x1, y0, y1, wx, wy


def _blend(c00, c10, c01, c11, wx, wy):
    f0 = c00 + wx * (c10 - c00)
    f1 = c01 + wx * (c11 - c01)
    return f0 + wy * (f1 - f0)


def _sc_body(*refs):
    xs_hbm = refs[0]
    ys_hbm = refs[1]
    cb_small_hbm = refs[2:2 + N_SMALL]
    cb_big_hbm = refs[2 + N_SMALL:2 + NUM_LOD]
    out_f = refs[2 + NUM_LOD]
    scr = refs[3 + NUM_LOD:]
    cbv = scr[0:N_SMALL]
    xv = scr[N_SMALL]
    yv = scr[N_SMALL + 1]
    ovs = scr[N_SMALL + 2:N_SMALL + 4]
    idxr = scr[N_SMALL + 4:N_SMALL + 4 + 4 * N_BIG]
    rowr = scr[N_SMALL + 4 + 4 * N_BIG:N_SMALL + 4 + 8 * N_BIG]
    sem = scr[N_SMALL + 4 + 8 * N_BIG]
    osem = scr[N_SMALL + 5 + 8 * N_BIG]

    sid = lax.axis_index("s")
    wid = sid * NC + lax.axis_index("c")
    for i in range(N_SMALL):
        pltpu.sync_copy(cb_small_hbm[i], cbv[i])

    base0 = wid * PPW
    lane = lax.iota(jnp.int32, LANES)
    z16 = lane * 0
    o16 = z16 + 1

    def chunk_compute(cc, base, ov):
        def load_xy(g):
            o = cc * C + g * LANES
            return xv[pl.ds(o, LANES)], yv[pl.ds(o, LANES)]

        def idx_body(g, cr):
            xi, yi = load_xy(g)
            sl = pl.ds(g * LANES, LANES)
            for k, lod in enumerate(BIG_LODS):
                x0, x1, y0, y1, _, _ = _corners(xi, yi, lod)
                t0 = y0 * PRIME1_I32
                t1 = y1 * PRIME1_I32
                idxr[4 * k + 0][sl] = ((x0 ^ t0) & HASH_MASK) >> 2
                idxr[4 * k + 1][sl] = ((x1 ^ t0) & HASH_MASK) >> 2
                idxr[4 * k + 2][sl] = ((x0 ^ t1) & HASH_MASK) >> 2
                idxr[4 * k + 3][sl] = ((x1 ^ t1) & HASH_MASK) >> 2
            return cr

        lax.fori_loop(0, G, idx_body, 0)

        copies = []
        for k in range(N_BIG):
            for c4 in range(4):
                copies.append(
                    pltpu.async_copy(cb_big_hbm[k].at[idxr[4 * k + c4]],
                                     rowr[4 * k + c4], sem))

        def small_body(g, cr):
            xi, yi = load_xy(g)
            o = g * LANES
            for li, lod in enumerate(SMALL_LODS):
                x0, x1, y0, y1, wx, wy = _corners(xi, yi, lod)
                b0 = y0 * lod
                b1 = y1 * lod
                i00 = (x0 + b0) * 2
                i10 = (x1 + b0) * 2
                i01 = (x0 + b1) * 2
                i11 = (x1 + b1) * 2
                cb = cbv[li]
                c00a = plsc.load_gather(cb, [i00])
                c00b = plsc.load_gather(cb, [i00 + 1])
                c10a = plsc.load_gather(cb, [i10])
                c10b = plsc.load_gather(cb, [i10 + 1])
                c01a = plsc.load_gather(cb, [i01])
                c01b = plsc.load_gather(cb, [i01 + 1])
                c11a = plsc.load_gather(cb, [i11])
                c11b = plsc.load_gather(cb, [i11 + 1])
                fa = _blend(c00a, c10a, c01a, c11a, wx, wy)
                fb = _blend(c00b, c10b, c01b, c11b, wx, wy)
                ov[pl.ds((2 * li) * C + o, LANES)] = fa
                ov[pl.ds((2 * li + 1) * C + o, LANES)] = fb
            return cr

        lax.fori_loop(0, G, small_body, 0)

        for cp in copies:
            cp.wait()

        def big_body(g, cr):
            xi, yi = load_xy(g)
            o = g * LANES
            ridx = g * LANES + lane
            for k, lod in enumerate(BIG_LODS):
                x0, x1, y0, y1, wx, wy = _corners(xi, yi, lod)
                t0 = y0 * PRIME1_I32
                t1 = y1 * PRIME1_I32
                s00 = (((x0 ^ t0) & HASH_MASK) & (SR - 1)) * 2
                s10 = (((x1 ^ t0) & HASH_MASK) & (SR - 1)) * 2
                s01 = (((x0 ^ t1) & HASH_MASK) & (SR - 1)) * 2
                s11 = (((x1 ^ t1) & HASH_MASK) & (SR - 1)) * 2
                r00, r10, r01, r11 = (rowr[4 * k + j] for j in range(4))
                c00a = plsc.load_gather(r00, [ridx, s00])
                c00b = plsc.load_gather(r00, [ridx, s00 + 1])
                c10a = plsc.load_gather(r10, [ridx, s10])
                c10b = plsc.load_gather(r10, [ridx, s10 + 1])
                c01a = plsc.load_gather(r01, [ridx, s01])
                c01b = plsc.load_gather(r01, [ridx, s01 + 1])
                c11a = plsc.load_gather(r11, [ridx, s11])
                c11b = plsc.load_gather(r11, [ridx, s11 + 1])
                fa = _blend(c00a, c10a, c01a, c11a, wx, wy)
                fb = _blend(c00b, c10b, c01b, c11b, wx, wy)
                li = N_SMALL + k
                ov[pl.ds((2 * li) * C + o, LANES)] = fa
                ov[pl.ds((2 * li + 1) * C + o, LANES)] = fb
            return cr

        lax.fori_loop(0, G, big_body, 0)

        half = (N_POINTS // C) * (8 * C)
        pltpu.async_copy(ov.at[pl.ds(0, 8 * C)],
                         out_f.at[pl.ds(base * 8, 8 * C)], osem)
        pltpu.async_copy(ov.at[pl.ds(8 * C, 8 * C)],
                         out_f.at[pl.ds(half + base * 8, 8 * C)], osem)

    def drain_outs():
        for _ in range(2):
            pltpu.make_async_copy(ovs[0].at[pl.ds(0, 8 * C)],
                                  out_f.at[pl.ds(0, 8 * C)], osem).wait()
            pltpu.make_async_copy(ovs[1].at[pl.ds(0, 8 * C)],
                                  out_f.at[pl.ds(0, 8 * C)], osem).wait()

    def step_body(i, carry):
        base = base0 + i * (2 * C)
        pltpu.sync_copy(xs_hbm.at[pl.ds(base, 2 * C)], xv)
        pltpu.sync_copy(ys_hbm.at[pl.ds(base, 2 * C)], yv)

        @pl.when(i > 0)
        def _():
            drain_outs()

        chunk_compute(0, base, ovs[0])
        chunk_compute(1, base + C, ovs[1])
        return carry

    lax.fori_loop(0, NCH // 2, step_body, 0)
    drain_outs()


_hash_grid_sc = functools.partial(
    pl.kernel,
    mesh=_mesh,
    out_type=jax.ShapeDtypeStruct((N_POINTS * 16,), jnp.float32),
    scratch_types=_scratch,
    compiler_params=pltpu.CompilerParams(needs_layout_passes=False,
                                         use_tc_tiling_on_sc=False),
)()(_sc_body)


def kernel(coords, codebook_0, codebook_1, codebook_2, codebook_3,
           codebook_4, codebook_5, codebook_6, codebook_7):
    cbs = (codebook_0, codebook_1, codebook_2, codebook_3,
           codebook_4, codebook_5, codebook_6, codebook_7)
    small = [cb.reshape(-1) for cb, l in zip(cbs, LODS) if l * l <= CODEBOOK_SIZE]
    big = [cb.reshape(BIG_ROWS, 2 * SR)
           for cb, l in zip(cbs, LODS) if l * l > CODEBOOK_SIZE]
    out = _hash_grid_sc(coords[:, 0], coords[:, 1], *small, *big)
    nb = N_POINTS // C
    return (out.reshape(2, nb, 8, C).transpose(1, 3, 0, 2)
            .reshape(N_POINTS, 16))
